# baseline (speedup 1.0000x reference)
import jax
import jax.numpy as jnp
from jax.experimental import pallas as pl
from jax.experimental.pallas import tpu as pltpu

B = 4096
D = 256
T = 16
K = 4
DFF = 512
NT = 4
TPS = T // NT


def _fused(op_idx_ref, a_ref, b_ref, c_ref, op_embed_ref, W_in_ref, b_in_ref,
           mix_W_ref, mix_b_ref, W_g_ref, W1_ref, b1_ref, W2_ref, b2_ref,
           W_h1_ref, b_h1_ref, W_h2_ref, b_h2_ref,
           result_ref, probs_ref, aux_ref,
           x_s, gates_s, acc_s, aux_s, t1_s, t2_s):
    s = pl.program_id(0)
    f32 = jnp.float32

    @pl.when(s == 0)
    def _featurize():
        col8 = jax.lax.broadcasted_iota(jnp.int32, (1, 8), 1)
        onehot = (op_idx_ref[...] == col8).astype(f32)
        abits = ((a_ref[...] >> col8) & 1).astype(f32)
        bbits = ((b_ref[...] >> col8) & 1).astype(f32)
        cf = c_ref[...].astype(f32)

        W_op = jnp.dot(op_embed_ref[...], W_in_ref[0:32, :],
                       preferred_element_type=f32)
        x = (jnp.dot(onehot, W_op, preferred_element_type=f32)
             + jnp.dot(abits, W_in_ref[32:40, :], preferred_element_type=f32)
             + jnp.dot(bbits, W_in_ref[40:48, :], preferred_element_type=f32)
             + cf * W_in_ref[48:49, :]
             + b_in_ref[...])

        x = x + jnp.dot(jnp.tanh(x), mix_W_ref[...],
                        preferred_element_type=f32) + mix_b_ref[...]

        ones16 = jnp.ones((T, 1), f32)
        logits = jnp.dot(x, W_g_ref[...], preferred_element_type=f32)
        m = jnp.max(logits, axis=1, keepdims=True)
        e = jnp.exp(logits - m)
        p = e / jnp.dot(e, ones16, preferred_element_type=f32)

        masked = p
        selected = jnp.zeros(p.shape, jnp.bool_)
        lane16 = jax.lax.broadcasted_iota(jnp.int32, (1, T), 1)
        for _ in range(K):
            mx = jnp.max(masked, axis=1, keepdims=True)
            is_max = masked == mx
            minidx = jnp.min(jnp.where(is_max, lane16, T), axis=1,
                             keepdims=True)
            sel = lane16 == minidx
            selected = selected | sel
            masked = jnp.where(sel, -1.0, masked)
        gk = jnp.where(selected, p, 0.0)
        gates = gk / (jnp.dot(gk, ones16, preferred_element_type=f32) + 1e-9)

        x_s[...] = x
        gates_s[...] = gates
        probs_ref[...] = p

        onesB = jnp.ones((1, B), f32)
        imp = jnp.dot(onesB, p, preferred_element_type=f32) / B
        load = jnp.dot(onesB, gates, preferred_element_type=f32) / B
        diversity = 0.01 * T * jnp.dot(imp * load, ones16,
                                       preferred_element_type=f32)
        psum = jnp.dot(jnp.dot(onesB, p * (1.0 - p),
                               preferred_element_type=f32), ones16,
                       preferred_element_type=f32)
        sparsity = 0.005 * psum / B
        aux_s[...] = diversity + sparsity

    x = x_s[...]
    gates = gates_s[...]
    row16 = jax.lax.broadcasted_iota(jnp.int32, (T, 1), 0)
    acc = jnp.zeros((B, D), f32)
    tern1 = jnp.zeros((1, 1), f32)
    tern2 = jnp.zeros((1, 1), f32)
    for j in range(TPS):
        t = s * TPS + j
        h = jnp.maximum(jnp.dot(x, W1_ref[j], preferred_element_type=f32)
                        + b1_ref[j], 0.0)
        eo = jnp.dot(h, W2_ref[j], preferred_element_type=f32)
        g = jnp.dot(gates, (row16 == t).astype(f32),
                    preferred_element_type=f32)
        acc = acc + g * eo
        tw1 = jnp.abs(jnp.tanh(W1_ref[j]))
        tw2 = jnp.abs(jnp.tanh(W2_ref[j]))
        tern1 = tern1 + jnp.sum(tw1 * (1.0 - tw1), axis=(0, 1), keepdims=True)
        tern2 = tern2 + jnp.sum(tw2 * (1.0 - tw2), axis=(0, 1), keepdims=True)

    @pl.when(s == 0)
    def _init_acc():
        acc_s[...] = acc
        t1_s[...] = tern1
        t2_s[...] = tern2

    @pl.when(s > 0)
    def _accum_acc():
        acc_s[...] += acc
        t1_s[...] += tern1
        t2_s[...] += tern2

    @pl.when(s == NT - 1)
    def _head():
        out = acc_s[...] + jnp.dot(gates_s[...], b2_ref[...],
                                   preferred_element_type=f32)
        r = jnp.maximum(jnp.dot(out, W_h1_ref[...], preferred_element_type=f32)
                        + b_h1_ref[...], 0.0)
        z = jnp.dot(r, W_h2_ref[...], preferred_element_type=f32) + b_h2_ref[...]
        result_ref[...] = 1.0 / (1.0 + jnp.exp(-z))
        ternary = 0.01 * (t1_s[...] / (T * D * DFF) + t2_s[...] / (T * DFF * D))
        aux_ref[...] = aux_s[...] + ternary


def kernel(op_idx, a, b, c, op_embed, W_in, b_in, mix_W, mix_b, W_g,
           W1, b1, W2, b2, W_h1, b_h1, W_h2, b_h2):
    op_idx2 = op_idx.astype(jnp.int32).reshape(B, 1)
    a2 = a.astype(jnp.int32).reshape(B, 1)
    b2d = b.astype(jnp.int32).reshape(B, 1)
    c2 = c.astype(jnp.int32).reshape(B, 1)
    b_in2 = b_in.reshape(1, D)
    b1_3 = b1.reshape(T, 1, DFF)
    mix_b2 = mix_b.reshape(1, D)
    b_h1_2 = b_h1.reshape(1, 64)
    b_h2_2 = b_h2.reshape(1, 8)

    full2 = lambda i: (0, 0)
    tiles3 = lambda i: (i, 0, 0)

    result, probs, aux = pl.pallas_call(
        _fused,
        grid=(NT,),
        in_specs=[
            pl.BlockSpec((B, 1), full2),
            pl.BlockSpec((B, 1), full2),
            pl.BlockSpec((B, 1), full2),
            pl.BlockSpec((B, 1), full2),
            pl.BlockSpec((8, 32), full2),
            pl.BlockSpec((49, D), full2),
            pl.BlockSpec((1, D), full2),
            pl.BlockSpec((D, D), full2),
            pl.BlockSpec((1, D), full2),
            pl.BlockSpec((D, T), full2),
            pl.BlockSpec((TPS, D, DFF), tiles3),
            pl.BlockSpec((TPS, 1, DFF), tiles3),
            pl.BlockSpec((TPS, DFF, D), tiles3),
            pl.BlockSpec((T, D), full2),
            pl.BlockSpec((D, 64), full2),
            pl.BlockSpec((1, 64), full2),
            pl.BlockSpec((64, 8), full2),
            pl.BlockSpec((1, 8), full2),
        ],
        out_specs=[
            pl.BlockSpec((B, 8), full2),
            pl.BlockSpec((B, T), full2),
            pl.BlockSpec((1, 1), full2),
        ],
        out_shape=[
            jax.ShapeDtypeStruct((B, 8), jnp.float32),
            jax.ShapeDtypeStruct((B, T), jnp.float32),
            jax.ShapeDtypeStruct((1, 1), jnp.float32),
        ],
        scratch_shapes=[
            pltpu.VMEM((B, D), jnp.float32),
            pltpu.VMEM((B, T), jnp.float32),
            pltpu.VMEM((B, D), jnp.float32),
            pltpu.VMEM((1, 1), jnp.float32),
            pltpu.VMEM((1, 1), jnp.float32),
            pltpu.VMEM((1, 1), jnp.float32),
        ],
        compiler_params=pltpu.CompilerParams(
            dimension_semantics=("arbitrary",),
        ),
    )(op_idx2, a2, b2d, c2, op_embed, W_in, b_in2, mix_W, mix_b2, W_g,
      W1, b1_3, W2, b2, W_h1, b_h1_2, W_h2, b_h2_2)

    return result, probs, aux.reshape(())


# speedup vs baseline: 1.1763x; 1.1763x over previous
import jax
import jax.numpy as jnp
from jax.experimental import pallas as pl
from jax.experimental.pallas import tpu as pltpu

B = 4096
D = 256
T = 16
K = 4
DFF = 512
NT = 4
TPS = T // NT


def _fused(op_idx_ref, a_ref, b_ref, c_ref, op_embed_ref, W_in_ref, b_in_ref,
           mix_W_ref, mix_b_ref, W_g_ref, W1_ref, b1_ref, W2_ref, b2_ref,
           W_h1_ref, b_h1_ref, W_h2_ref, b_h2_ref,
           result_ref, probs_ref, aux_ref,
           x_s, gates_s, acc_s, aux_s, t1_s, t2_s):
    s = pl.program_id(0)
    f32 = jnp.float32

    @pl.when(s == 0)
    def _featurize():
        col8 = jax.lax.broadcasted_iota(jnp.int32, (1, 8), 1)
        onehot = (op_idx_ref[...] == col8).astype(f32)
        abits = ((a_ref[...] >> col8) & 1).astype(f32)
        bbits = ((b_ref[...] >> col8) & 1).astype(f32)
        cf = c_ref[...].astype(f32)

        W_op = jnp.dot(op_embed_ref[...], W_in_ref[0:32, :],
                       preferred_element_type=f32)
        x = (jnp.dot(onehot, W_op, preferred_element_type=f32)
             + jnp.dot(abits, W_in_ref[32:40, :], preferred_element_type=f32)
             + jnp.dot(bbits, W_in_ref[40:48, :], preferred_element_type=f32)
             + cf * W_in_ref[48:49, :]
             + b_in_ref[...])

        x = x + jnp.dot(jnp.tanh(x), mix_W_ref[...],
                        preferred_element_type=f32) + mix_b_ref[...]

        logits = jnp.dot(x, W_g_ref[...], preferred_element_type=f32)
        m = jnp.max(logits, axis=1, keepdims=True)
        e = jnp.exp(logits - m)
        p = e / jnp.sum(e, axis=1, keepdims=True)

        masked = p
        selected = jnp.zeros(p.shape, jnp.bool_)
        lane16 = jax.lax.broadcasted_iota(jnp.int32, (1, T), 1)
        for _ in range(K):
            mx = jnp.max(masked, axis=1, keepdims=True)
            is_max = masked == mx
            minidx = jnp.min(jnp.where(is_max, lane16, T), axis=1,
                             keepdims=True)
            sel = lane16 == minidx
            selected = selected | sel
            masked = jnp.where(sel, -1.0, masked)
        gk = jnp.where(selected, p, 0.0)
        gates = gk / (jnp.sum(gk, axis=1, keepdims=True) + 1e-9)

        x_s[...] = x
        gates_s[...] = gates
        probs_ref[...] = p

        imp = jnp.sum(p, axis=0, keepdims=True) / B
        load = jnp.sum(gates, axis=0, keepdims=True) / B
        diversity = 0.01 * T * jnp.sum(imp * load, axis=(0, 1), keepdims=True)
        sparsity = 0.005 * jnp.sum(p * (1.0 - p), axis=(0, 1),
                                   keepdims=True) / B
        aux_s[...] = diversity + sparsity

    x = x_s[...]
    gates = gates_s[...]
    lane16 = jax.lax.broadcasted_iota(jnp.int32, (1, T), 1)
    acc = jnp.zeros((B, D), f32)
    tern1 = jnp.zeros((1, 1), f32)
    tern2 = jnp.zeros((1, 1), f32)
    for j in range(TPS):
        t = s * TPS + j
        h = jnp.maximum(jnp.dot(x, W1_ref[j], preferred_element_type=f32)
                        + b1_ref[j], 0.0)
        eo = jnp.dot(h, W2_ref[j], preferred_element_type=f32)
        g = jnp.sum(gates * (lane16 == t).astype(f32), axis=1, keepdims=True)
        acc = acc + g * eo
        tw1 = jnp.abs(jnp.tanh(W1_ref[j]))
        tw2 = jnp.abs(jnp.tanh(W2_ref[j]))
        tern1 = tern1 + jnp.sum(tw1 * (1.0 - tw1), axis=(0, 1), keepdims=True)
        tern2 = tern2 + jnp.sum(tw2 * (1.0 - tw2), axis=(0, 1), keepdims=True)

    @pl.when(s == 0)
    def _init_acc():
        acc_s[...] = acc
        t1_s[...] = tern1
        t2_s[...] = tern2

    @pl.when(s > 0)
    def _accum_acc():
        acc_s[...] += acc
        t1_s[...] += tern1
        t2_s[...] += tern2

    @pl.when(s == NT - 1)
    def _head():
        out = acc_s[...] + jnp.dot(gates_s[...], b2_ref[...],
                                   preferred_element_type=f32)
        r = jnp.maximum(jnp.dot(out, W_h1_ref[...], preferred_element_type=f32)
                        + b_h1_ref[...], 0.0)
        z = jnp.dot(r, W_h2_ref[...], preferred_element_type=f32) + b_h2_ref[...]
        result_ref[...] = 1.0 / (1.0 + jnp.exp(-z))
        ternary = 0.01 * (t1_s[...] / (T * D * DFF) + t2_s[...] / (T * DFF * D))
        aux_ref[...] = aux_s[...] + ternary


def kernel(op_idx, a, b, c, op_embed, W_in, b_in, mix_W, mix_b, W_g,
           W1, b1, W2, b2, W_h1, b_h1, W_h2, b_h2):
    op_idx2 = op_idx.astype(jnp.int32).reshape(B, 1)
    a2 = a.astype(jnp.int32).reshape(B, 1)
    b2d = b.astype(jnp.int32).reshape(B, 1)
    c2 = c.astype(jnp.int32).reshape(B, 1)
    b_in2 = b_in.reshape(1, D)
    b1_3 = b1.reshape(T, 1, DFF)
    mix_b2 = mix_b.reshape(1, D)
    b_h1_2 = b_h1.reshape(1, 64)
    b_h2_2 = b_h2.reshape(1, 8)

    full2 = lambda i: (0, 0)
    tiles3 = lambda i: (i, 0, 0)

    result, probs, aux = pl.pallas_call(
        _fused,
        grid=(NT,),
        in_specs=[
            pl.BlockSpec((B, 1), full2),
            pl.BlockSpec((B, 1), full2),
            pl.BlockSpec((B, 1), full2),
            pl.BlockSpec((B, 1), full2),
            pl.BlockSpec((8, 32), full2),
            pl.BlockSpec((49, D), full2),
            pl.BlockSpec((1, D), full2),
            pl.BlockSpec((D, D), full2),
            pl.BlockSpec((1, D), full2),
            pl.BlockSpec((D, T), full2),
            pl.BlockSpec((TPS, D, DFF), tiles3),
            pl.BlockSpec((TPS, 1, DFF), tiles3),
            pl.BlockSpec((TPS, DFF, D), tiles3),
            pl.BlockSpec((T, D), full2),
            pl.BlockSpec((D, 64), full2),
            pl.BlockSpec((1, 64), full2),
            pl.BlockSpec((64, 8), full2),
            pl.BlockSpec((1, 8), full2),
        ],
        out_specs=[
            pl.BlockSpec((B, 8), full2),
            pl.BlockSpec((B, T), full2),
            pl.BlockSpec((1, 1), full2),
        ],
        out_shape=[
            jax.ShapeDtypeStruct((B, 8), jnp.float32),
            jax.ShapeDtypeStruct((B, T), jnp.float32),
            jax.ShapeDtypeStruct((1, 1), jnp.float32),
        ],
        scratch_shapes=[
            pltpu.VMEM((B, D), jnp.float32),
            pltpu.VMEM((B, T), jnp.float32),
            pltpu.VMEM((B, D), jnp.float32),
            pltpu.VMEM((1, 1), jnp.float32),
            pltpu.VMEM((1, 1), jnp.float32),
            pltpu.VMEM((1, 1), jnp.float32),
        ],
        compiler_params=pltpu.CompilerParams(
            dimension_semantics=("arbitrary",),
        ),
    )(op_idx2, a2, b2d, c2, op_embed, W_in, b_in2, mix_W, mix_b2, W_g,
      W1, b1_3, W2, b2, W_h1, b_h1_2, W_h2, b_h2_2)

    return result, probs, aux.reshape(())
